# baseline (speedup 1.0000x reference)
import jax
import jax.numpy as jnp
from jax import lax
from jax.experimental import pallas as pl
from jax.experimental.pallas import tpu as pltpu
from jax.experimental.pallas import tpu_sc as plsc

_ROW = 32
_NW = 32
_CH = 1024
_G = _CH // 32


def _batcher_pairs(n):
    pairs = []

    def merge(lo, m, r):
        step = r * 2
        if step < m:
            merge(lo, m, step)
            merge(lo + r, m, step)
            for i in range(lo + r, lo + m - r, step):
                pairs.append((i, i + r))
        else:
            pairs.append((lo, lo + r))

    def sort(lo, m):
        if m > 1:
            k = m // 2
            sort(lo, k)
            sort(lo + k, k)
            merge(lo, m, 1)

    sort(0, n)
    return pairs


_PAIRS = _batcher_pairs(_ROW)


def _sorted16(buf, base, rowoff):
    vs = [plsc.load_gather(buf, [rowoff + (base + k)]) for k in range(_ROW)]
    for (i, j) in _PAIRS:
        a, b = vs[i], vs[j]
        vs[i] = jnp.minimum(a, b)
        vs[j] = jnp.maximum(a, b)
    eqs = [vs[k] == vs[k - 1] for k in range(1, _ROW)]
    anydup = eqs[0]
    for e in eqs[1:]:
        anydup = anydup | e
    return vs, eqs, anydup


def _scan16(vs, eqs):
    run = jnp.ones((16,), jnp.int32)
    best = run
    bestv = vs[0]
    for k in range(1, _ROW):
        run = run * eqs[k - 1].astype(jnp.int32) + 1
        bt = run > best
        best = jnp.maximum(run, best)
        bestv = jnp.where(bt, vs[k], bestv)
    return bestv


def _sc_body(x_hbm, o_hbm, buf, obuf):
    n = o_hbm.shape[0]
    rpw = n // _NW
    wid = lax.axis_index("s") * 2 + lax.axis_index("c")
    base_row = wid * rpw
    rowoff = lax.iota(jnp.int32, 16) * _ROW

    def chunk(c, _):
        row0 = base_row + c * _CH
        pltpu.sync_copy(x_hbm.at[pl.ds(row0 * _ROW, _CH * _ROW)], buf)

        def group(g, _):
            base = g * (32 * _ROW)
            vs0, eqs0, dup0 = _sorted16(buf, base, rowoff)
            vs1, eqs1, dup1 = _sorted16(buf, base + 16 * _ROW, rowoff)
            b0, b1 = lax.cond(
                jnp.any(dup0 | dup1),
                lambda: (_scan16(vs0, eqs0), _scan16(vs1, eqs1)),
                lambda: (vs0[0], vs1[0]),
            )
            obuf[pl.ds(g * 32, 16)] = b0
            obuf[pl.ds(g * 32 + 16, 16)] = b1
            return 0

        lax.fori_loop(0, _G, group, 0)
        pltpu.sync_copy(obuf, o_hbm.at[pl.ds(row0, _CH)])
        return 0

    lax.fori_loop(0, rpw // _CH, chunk, 0)


def kernel(x):
    n = x.shape[0]
    xf = x.reshape(n * _ROW)
    out = pl.kernel(
        _sc_body,
        out_type=jax.ShapeDtypeStruct((n,), jnp.float32),
        mesh=plsc.VectorSubcoreMesh(core_axis_name="c", subcore_axis_name="s"),
        scratch_types=[
            pltpu.VMEM((_CH * _ROW,), jnp.float32),
            pltpu.VMEM((_CH,), jnp.float32),
        ],
        compiler_params=pltpu.CompilerParams(needs_layout_passes=False),
    )(xf)
    return out


# speedup vs baseline: 1.0603x; 1.0603x over previous
import jax
import jax.numpy as jnp
from jax import lax
from jax.experimental import pallas as pl
from jax.experimental.pallas import tpu as pltpu
from jax.experimental.pallas import tpu_sc as plsc

_ROW = 32
_NW = 32
_CH = 1024
_G = _CH // 16


def _batcher_pairs(n):
    pairs = []

    def merge(lo, m, r):
        step = r * 2
        if step < m:
            merge(lo, m, step)
            merge(lo + r, m, step)
            for i in range(lo + r, lo + m - r, step):
                pairs.append((i, i + r))
        else:
            pairs.append((lo, lo + r))

    def sort(lo, m):
        if m > 1:
            k = m // 2
            sort(lo, k)
            sort(lo + k, k)
            merge(lo, m, 1)

    sort(0, n)
    return pairs


_PAIRS = _batcher_pairs(_ROW)


def _mode16(buf, base, rowoff):
    vs = [plsc.load_gather(buf, [rowoff + (base + k)]) for k in range(_ROW)]
    for (i, j) in _PAIRS:
        a, b = vs[i], vs[j]
        vs[i] = jnp.minimum(a, b)
        vs[j] = jnp.maximum(a, b)
    anydup = vs[1] == vs[0]
    for k in range(2, _ROW):
        anydup = anydup | (vs[k] == vs[k - 1])

    def with_scan():
        run = jnp.ones((16,), jnp.int32)
        best = run
        bestv = vs[0]
        for k in range(1, _ROW):
            run = run * (vs[k] == vs[k - 1]).astype(jnp.int32) + 1
            bt = run > best
            best = jnp.maximum(run, best)
            bestv = jnp.where(bt, vs[k], bestv)
        return bestv

    return lax.cond(jnp.any(anydup), with_scan, lambda: vs[0])


def _sc_body(x_hbm, o_hbm, buf0, buf1, obuf, sem0, sem1):
    n = o_hbm.shape[0]
    rpw = n // _NW
    nch = rpw // _CH
    wid = lax.axis_index("s") * 2 + lax.axis_index("c")
    base_row = wid * rpw
    rowoff = lax.iota(jnp.int32, 16) * _ROW

    def src(c):
        return x_hbm.at[pl.ds((base_row + c * _CH) * _ROW, _CH * _ROW)]

    def compute(buf, c):
        def group(g, _):
            obuf[pl.ds(g * 16, 16)] = _mode16(buf, g * (16 * _ROW), rowoff)
            return 0

        lax.fori_loop(0, _G, group, 0)
        pltpu.sync_copy(obuf, o_hbm.at[pl.ds(base_row + c * _CH, _CH)])

    pltpu.async_copy(src(0), buf0, sem0)
    pltpu.async_copy(src(1), buf1, sem1)

    def pair(cc, _):
        c0 = cc * 2
        pltpu.make_async_copy(src(c0), buf0, sem0).wait()
        compute(buf0, c0)

        @pl.when(cc < nch // 2 - 1)
        def _():
            pltpu.async_copy(src(c0 + 2), buf0, sem0)

        pltpu.make_async_copy(src(c0 + 1), buf1, sem1).wait()
        compute(buf1, c0 + 1)

        @pl.when(cc < nch // 2 - 1)
        def _():
            pltpu.async_copy(src(c0 + 3), buf1, sem1)

        return 0

    lax.fori_loop(0, nch // 2, pair, 0)


def kernel(x):
    n = x.shape[0]
    xf = x.reshape(n * _ROW)
    out = pl.kernel(
        _sc_body,
        out_type=jax.ShapeDtypeStruct((n,), jnp.float32),
        mesh=plsc.VectorSubcoreMesh(core_axis_name="c", subcore_axis_name="s"),
        scratch_types=[
            pltpu.VMEM((_CH * _ROW,), jnp.float32),
            pltpu.VMEM((_CH * _ROW,), jnp.float32),
            pltpu.VMEM((_CH,), jnp.float32),
            pltpu.SemaphoreType.DMA,
            pltpu.SemaphoreType.DMA,
        ],
        compiler_params=pltpu.CompilerParams(needs_layout_passes=False),
    )(xf)
    return out
